# baseline (speedup 1.0000x reference)
import functools

import jax
import jax.numpy as jnp
from jax import lax
from jax.experimental import pallas as pl
from jax.experimental.pallas import tpu as pltpu
from jax.experimental.pallas import tpu_sc as plsc

MAX_SPEAKERS = 1000000
EMBED_DIM = 64
HIDDEN_SIZE = 1024
BATCH = 16384
_L = 16
_NBUF = 6


def _make_sc_gather(B):
    info = plsc.get_sparse_core_info()
    NC, NS = info.num_cores, info.num_subcores
    NW = NC * NS
    b_per_w = B // NW
    mesh = plsc.VectorSubcoreMesh(core_axis_name="c", subcore_axis_name="s")

    @functools.partial(
        pl.kernel,
        mesh=mesh,
        out_type=jax.ShapeDtypeStruct((B, 2 * EMBED_DIM), jnp.float32),
        scratch_types=[
            pltpu.VMEM((b_per_w + _L,), jnp.int32),
            pltpu.VMEM((_NBUF, EMBED_DIM, 2 * EMBED_DIM), jnp.float32),
            pltpu.VMEM((b_per_w, 2 * EMBED_DIM), jnp.float32),
            pltpu.SemaphoreType.DMA,
        ],
        compiler_params=pltpu.CompilerParams(needs_layout_passes=False),
    )
    def gather_kernel(tableT_hbm, idx_hbm, x2_hbm, idx_v, slab_v, rows_v, sem):
        wid = lax.axis_index("s") * NC + lax.axis_index("c")
        base = wid * b_per_w
        pltpu.sync_copy(idx_hbm.at[pl.ds(base, b_per_w)], idx_v.at[pl.ds(0, b_per_w)])

        cvec = [lax.iota(jnp.int32, _L) + k * _L for k in range(4)]

        def fetch(scalar_id, buf):
            tc = lax.shift_right_logical(scalar_id, 7) * 128
            pltpu.async_copy(
                tableT_hbm.at[:, pl.ds(tc, 128)], slab_v.at[buf], sem
            )

        def extract(j, scalar_id, buf):
            lvec = jnp.full((_L,), scalar_id & 127, jnp.int32)
            for k in range(4):
                vals = plsc.load_gather(slab_v.at[buf], [cvec[k], lvec])
                rows_v[j, pl.ds(k * _L, _L)] = vals

        v0 = idx_v[pl.ds(0, _L)]
        for b in range(_NBUF):
            fetch(v0[b], b)

        n_blocks = b_per_w // _L

        def block_body(r, _):
            j0 = r * _L
            v_cur = idx_v[pl.ds(j0, _L)]
            v_nxt = idx_v[pl.ds(j0 + _L, _L)]
            for t in range(_L):
                j = j0 + t
                buf = (
                    lax.rem(j, _NBUF)
                    if _L % _NBUF
                    else t % _NBUF
                )
                pltpu.make_async_copy(
                    tableT_hbm.at[:, pl.ds(0, 128)], slab_v.at[buf], sem
                ).wait()
                extract(j, v_cur[t], buf)
                nid = (
                    v_cur[t + _NBUF] if t + _NBUF < _L else v_nxt[t + _NBUF - _L]
                )

                @pl.when(j + _NBUF < b_per_w)
                def _():
                    fetch(nid, buf)

            return ()

        lax.fori_loop(0, n_blocks, block_body, (), unroll=False)
        pltpu.sync_copy(rows_v, x2_hbm.at[pl.ds(base, b_per_w)])

    return gather_kernel


def _proj_body(x2_ref, wt_ref, b_ref, o_ref):
    o_ref[...] = (
        lax.dot_general(
            x2_ref[:, :EMBED_DIM],
            wt_ref[...],
            (((1,), (0,)), ((), ())),
            preferred_element_type=jnp.float32,
        )
        + b_ref[...]
    )


def _proj_body_acc(x2_ref, wt_ref, b_ref, prev_ref, o_ref):
    del prev_ref
    _proj_body(x2_ref, wt_ref, b_ref, o_ref)


_NCHUNK = 4
_BB = 1024


def _make_tc_proj(B, H, chunk):
    BC = B // _NCHUNK
    r0 = (chunk * BC) // _BB
    in_specs = [
        pl.BlockSpec((_BB, 2 * EMBED_DIM), lambda i: (i, 0)),
        pl.BlockSpec((EMBED_DIM, H), lambda i: (0, 0)),
        pl.BlockSpec((1, H), lambda i: (0, 0)),
    ]
    kwargs = {}
    body = _proj_body
    if chunk > 0:
        in_specs.append(pl.BlockSpec(memory_space=pl.ANY))
        kwargs["input_output_aliases"] = {3: 0}
        body = _proj_body_acc
    return pl.pallas_call(
        body,
        grid=(BC // _BB,),
        in_specs=in_specs,
        out_specs=pl.BlockSpec((_BB, H), lambda i: (r0 + i, 0)),
        out_shape=jax.ShapeDtypeStruct((B, H), jnp.float32),
        **kwargs,
    )


@jax.jit
def kernel(speaker_ids, table, W, b):
    ids = speaker_ids.astype(jnp.int32)
    tableT = table.T
    Wt = W.T
    b2 = b.reshape(1, HIDDEN_SIZE)
    BC = BATCH // _NCHUNK
    gather = _make_sc_gather(BC)
    x2s = [
        gather(tableT, lax.slice(ids, (c * BC,), ((c + 1) * BC,)))
        for c in range(_NCHUNK)
    ]
    out = _make_tc_proj(BATCH, HIDDEN_SIZE, 0)(x2s[0], Wt, b2)
    for c in range(1, _NCHUNK):
        out = _make_tc_proj(BATCH, HIDDEN_SIZE, c)(x2s[c], Wt, b2, out)
    return out


# speedup vs baseline: 1.0356x; 1.0356x over previous
import functools

import jax
import jax.numpy as jnp
from jax import lax
from jax.experimental import pallas as pl
from jax.experimental.pallas import tpu as pltpu
from jax.experimental.pallas import tpu_sc as plsc

MAX_SPEAKERS = 1000000
EMBED_DIM = 64
HIDDEN_SIZE = 1024
BATCH = 16384
_L = 16
_NBUF = 6


def _make_sc_gather(B):
    info = plsc.get_sparse_core_info()
    NC, NS = info.num_cores, info.num_subcores
    NW = NC * NS
    b_per_w = B // NW
    mesh = plsc.VectorSubcoreMesh(core_axis_name="c", subcore_axis_name="s")

    @functools.partial(
        pl.kernel,
        mesh=mesh,
        out_type=jax.ShapeDtypeStruct((B, 2 * EMBED_DIM), jnp.float32),
        scratch_types=[
            pltpu.VMEM((b_per_w + _L,), jnp.int32),
            pltpu.VMEM((_NBUF, EMBED_DIM, 2 * EMBED_DIM), jnp.float32),
            pltpu.VMEM((b_per_w, 2 * EMBED_DIM), jnp.float32),
            pltpu.SemaphoreType.DMA,
        ],
        compiler_params=pltpu.CompilerParams(needs_layout_passes=False),
    )
    def gather_kernel(tableT_hbm, idx_hbm, x2_hbm, idx_v, slab_v, rows_v, sem):
        wid = lax.axis_index("s") * NC + lax.axis_index("c")
        base = wid * b_per_w
        pltpu.sync_copy(idx_hbm.at[pl.ds(base, b_per_w)], idx_v.at[pl.ds(0, b_per_w)])

        cvec = [lax.iota(jnp.int32, _L) + k * _L for k in range(4)]

        def fetch(scalar_id, buf):
            tc = lax.shift_right_logical(scalar_id, 7) * 128
            pltpu.async_copy(
                tableT_hbm.at[:, pl.ds(tc, 128)], slab_v.at[buf], sem
            )

        def extract(j, scalar_id, buf):
            lvec = jnp.full((_L,), scalar_id & 127, jnp.int32)
            for k in range(4):
                vals = plsc.load_gather(slab_v.at[buf], [cvec[k], lvec])
                rows_v[j, pl.ds(k * _L, _L)] = vals

        v0 = idx_v[pl.ds(0, _L)]
        for b in range(_NBUF):
            fetch(v0[b], b)

        n_blocks = b_per_w // _L

        def block_body(r, _):
            j0 = r * _L
            v_cur = idx_v[pl.ds(j0, _L)]
            v_nxt = idx_v[pl.ds(j0 + _L, _L)]
            for t in range(_L):
                j = j0 + t
                buf = (
                    lax.rem(j, _NBUF)
                    if _L % _NBUF
                    else t % _NBUF
                )
                pltpu.make_async_copy(
                    tableT_hbm.at[:, pl.ds(0, 128)], slab_v.at[buf], sem
                ).wait()
                extract(j, v_cur[t], buf)
                nid = (
                    v_cur[t + _NBUF] if t + _NBUF < _L else v_nxt[t + _NBUF - _L]
                )

                @pl.when(j + _NBUF < b_per_w)
                def _():
                    fetch(nid, buf)

            return ()

        lax.fori_loop(0, n_blocks, block_body, (), unroll=False)
        pltpu.sync_copy(rows_v, x2_hbm.at[pl.ds(base, b_per_w)])

    return gather_kernel


def _proj_body(x2_ref, wt_ref, b_ref, o_ref):
    o_ref[...] = (
        lax.dot_general(
            x2_ref[:, :EMBED_DIM],
            wt_ref[...],
            (((1,), (0,)), ((), ())),
            preferred_element_type=jnp.float32,
        )
        + b_ref[...]
    )


def _proj_body_acc(x2_ref, wt_ref, b_ref, prev_ref, o_ref):
    del prev_ref
    _proj_body(x2_ref, wt_ref, b_ref, o_ref)


_NCHUNK = 2
_BB = 1024


def _make_tc_proj(B, H, chunk):
    BC = B // _NCHUNK
    r0 = (chunk * BC) // _BB
    in_specs = [
        pl.BlockSpec((_BB, 2 * EMBED_DIM), lambda i: (i, 0)),
        pl.BlockSpec((EMBED_DIM, H), lambda i: (0, 0)),
        pl.BlockSpec((1, H), lambda i: (0, 0)),
    ]
    kwargs = {}
    body = _proj_body
    if chunk > 0:
        in_specs.append(pl.BlockSpec(memory_space=pl.ANY))
        kwargs["input_output_aliases"] = {3: 0}
        body = _proj_body_acc
    return pl.pallas_call(
        body,
        grid=(BC // _BB,),
        in_specs=in_specs,
        out_specs=pl.BlockSpec((_BB, H), lambda i: (r0 + i, 0)),
        out_shape=jax.ShapeDtypeStruct((B, H), jnp.float32),
        **kwargs,
    )


@jax.jit
def kernel(speaker_ids, table, W, b):
    ids = speaker_ids.astype(jnp.int32)
    tableT = table.T
    Wt = W.T
    b2 = b.reshape(1, HIDDEN_SIZE)
    BC = BATCH // _NCHUNK
    gather = _make_sc_gather(BC)
    x2s = [
        gather(tableT, lax.slice(ids, (c * BC,), ((c + 1) * BC,)))
        for c in range(_NCHUNK)
    ]
    out = _make_tc_proj(BATCH, HIDDEN_SIZE, 0)(x2s[0], Wt, b2)
    for c in range(1, _NCHUNK):
        out = _make_tc_proj(BATCH, HIDDEN_SIZE, c)(x2s[c], Wt, b2, out)
    return out


# speedup vs baseline: 1.0520x; 1.0159x over previous
import functools

import jax
import jax.numpy as jnp
from jax import lax
from jax.experimental import pallas as pl
from jax.experimental.pallas import tpu as pltpu
from jax.experimental.pallas import tpu_sc as plsc

MAX_SPEAKERS = 1000000
EMBED_DIM = 64
HIDDEN_SIZE = 1024
BATCH = 16384
_L = 16
_NBUF = 6


def _make_sc_gather(B):
    info = plsc.get_sparse_core_info()
    NC, NS = info.num_cores, info.num_subcores
    NW = NC * NS
    b_per_w = B // NW
    mesh = plsc.VectorSubcoreMesh(core_axis_name="c", subcore_axis_name="s")

    @functools.partial(
        pl.kernel,
        mesh=mesh,
        out_type=jax.ShapeDtypeStruct((B, 2 * EMBED_DIM), jnp.float32),
        scratch_types=[
            pltpu.VMEM((b_per_w + _L,), jnp.int32),
            pltpu.VMEM((_NBUF, EMBED_DIM, 2 * EMBED_DIM), jnp.float32),
            pltpu.VMEM((b_per_w, 2 * EMBED_DIM), jnp.float32),
            pltpu.SemaphoreType.DMA,
        ],
        compiler_params=pltpu.CompilerParams(needs_layout_passes=False),
    )
    def gather_kernel(tableT_hbm, idx_hbm, x2_hbm, idx_v, slab_v, rows_v, sem):
        wid = lax.axis_index("s") * NC + lax.axis_index("c")
        base = wid * b_per_w
        pltpu.sync_copy(idx_hbm.at[pl.ds(base, b_per_w)], idx_v.at[pl.ds(0, b_per_w)])

        cvec = [lax.iota(jnp.int32, _L) + k * _L for k in range(4)]

        def fetch(scalar_id, buf):
            tc = lax.shift_right_logical(scalar_id, 7) * 128
            pltpu.async_copy(
                tableT_hbm.at[:, pl.ds(tc, 128)], slab_v.at[buf], sem
            )

        def extract(j, scalar_id, buf):
            lvec = jnp.full((_L,), scalar_id & 127, jnp.int32)
            for k in range(4):
                vals = plsc.load_gather(slab_v.at[buf], [cvec[k], lvec])
                rows_v[j, pl.ds(k * _L, _L)] = vals

        v0 = idx_v[pl.ds(0, _L)]
        for b in range(_NBUF):
            fetch(v0[b], b)

        n_blocks = b_per_w // _L

        def block_body(r, _):
            j0 = r * _L
            v_cur = idx_v[pl.ds(j0, _L)]
            v_nxt = idx_v[pl.ds(j0 + _L, _L)]
            for t in range(_L):
                j = j0 + t
                buf = (
                    lax.rem(j, _NBUF)
                    if _L % _NBUF
                    else t % _NBUF
                )
                pltpu.make_async_copy(
                    tableT_hbm.at[:, pl.ds(0, 128)], slab_v.at[buf], sem
                ).wait()
                extract(j, v_cur[t], buf)
                nid = (
                    v_cur[t + _NBUF] if t + _NBUF < _L else v_nxt[t + _NBUF - _L]
                )

                @pl.when(j + _NBUF < b_per_w)
                def _():
                    fetch(nid, buf)

            return ()

        lax.fori_loop(0, n_blocks, block_body, (), unroll=False)
        pltpu.sync_copy(rows_v, x2_hbm.at[pl.ds(base, b_per_w)])

    return gather_kernel


def _proj_body(x2_ref, wt_ref, b_ref, o_ref):
    o_ref[...] = (
        lax.dot_general(
            x2_ref[:, :EMBED_DIM],
            wt_ref[...],
            (((1,), (0,)), ((), ())),
            preferred_element_type=jnp.float32,
        )
        + b_ref[...]
    )


def _proj_body_acc(x2_ref, wt_ref, b_ref, prev_ref, o_ref):
    del prev_ref
    _proj_body(x2_ref, wt_ref, b_ref, o_ref)


_NCHUNK = 1
_BB = 2048


def _make_tc_proj(B, H, chunk):
    BC = B // _NCHUNK
    r0 = (chunk * BC) // _BB
    in_specs = [
        pl.BlockSpec((_BB, 2 * EMBED_DIM), lambda i: (i, 0)),
        pl.BlockSpec((EMBED_DIM, H), lambda i: (0, 0)),
        pl.BlockSpec((1, H), lambda i: (0, 0)),
    ]
    kwargs = {}
    body = _proj_body
    if chunk > 0:
        in_specs.append(pl.BlockSpec(memory_space=pl.ANY))
        kwargs["input_output_aliases"] = {3: 0}
        body = _proj_body_acc
    return pl.pallas_call(
        body,
        grid=(BC // _BB,),
        in_specs=in_specs,
        out_specs=pl.BlockSpec((_BB, H), lambda i: (r0 + i, 0)),
        out_shape=jax.ShapeDtypeStruct((B, H), jnp.float32),
        **kwargs,
    )


@jax.jit
def kernel(speaker_ids, table, W, b):
    ids = speaker_ids.astype(jnp.int32)
    tableT = table.T
    Wt = W.T
    b2 = b.reshape(1, HIDDEN_SIZE)
    BC = BATCH // _NCHUNK
    gather = _make_sc_gather(BC)
    x2s = [
        gather(tableT, lax.slice(ids, (c * BC,), ((c + 1) * BC,)))
        for c in range(_NCHUNK)
    ]
    out = _make_tc_proj(BATCH, HIDDEN_SIZE, 0)(x2s[0], Wt, b2)
    for c in range(1, _NCHUNK):
        out = _make_tc_proj(BATCH, HIDDEN_SIZE, c)(x2s[c], Wt, b2, out)
    return out
